# Initial kernel scaffold; baseline (speedup 1.0000x reference)
import jax
import jax.numpy as jnp
from jax.experimental import pallas as pl


def kernel(x, edge_feat, Wr, Wself, Wm, We, gamma, beta, edge_index, edge_type, line_edge_index):
    raise NotImplementedError("write your pallas kernel here")


# speedup vs baseline: 2.2629x; 2.2629x over previous
import functools

import jax
import jax.numpy as jnp
from jax import lax
from jax.experimental import pallas as pl
from jax.experimental.pallas import tpu as pltpu
from jax.experimental.pallas import tpu_sc as plsc

N = 10000
E = 320000
D = 128
R = 7
L = 3

K = 128
NW = 32


@functools.lru_cache(maxsize=None)
def _make_sc_segsum(CH, C):
    CHP = CH + 8
    C1P = ((C + 1 + 16 + 7) // 8) * 8
    rounds = (C + NW - 1) // NW
    mesh = plsc.VectorSubcoreMesh(core_axis_name="c", subcore_axis_name="s")

    @functools.partial(
        pl.kernel, mesh=mesh,
        out_type=jax.ShapeDtypeStruct((C * CH, D), jnp.float32),
        scratch_types=[
            pltpu.VMEM((C1P,), jnp.int32),
            pltpu.VMEM((K,), jnp.int32),
            pltpu.VMEM((K,), jnp.int32),
            pltpu.VMEM((K,), jnp.int32),
            pltpu.VMEM((K, D), jnp.float32),
            pltpu.VMEM((CHP, D), jnp.float32),
            pltpu.VMEM_SHARED((16 * CHP, D), jnp.float32),
            pltpu.SemaphoreType.DMA,
        ],
    )
    def seg(table, g, locs, bounds, zeros, out,
            bnd_v, gidx_v, loc_v, sct_v, rows_v, zer_v, acc, sem):
        cid = lax.axis_index("c")
        sid = lax.axis_index("s")
        wid = sid * 2 + cid
        base = sid * CHP
        pltpu.sync_copy(bounds, bnd_v)
        pltpu.sync_copy(zeros, zer_v)

        def do_round(r, carry):
            c = r * NW + wid

            @pl.when(c < C)
            def _():
                bv = bnd_v[pl.ds(c, 16)]
                ts = bv[0]
                te = bv[1]
                a0 = lax.div(ts, 8) * 8
                nb = lax.div(te - a0 + (K - 1), K)
                pltpu.sync_copy(zer_v, acc.at[pl.ds(base, CHP)])

                def do_batch(b, carry2):
                    p = a0 + b * K
                    pltpu.sync_copy(g.at[pl.ds(p, K)], gidx_v)
                    pltpu.sync_copy(locs.at[pl.ds(p, K)], loc_v)
                    for j in range(K // 16):
                        pos = p + j * 16 + lax.iota(jnp.int32, 16)
                        lv = loc_v[pl.ds(j * 16, 16)]
                        ok = (pos >= ts) & (pos < te)
                        sct_v[pl.ds(j * 16, 16)] = jnp.where(
                            ok, lv + base, base + CH)
                    pltpu.async_copy(table.at[gidx_v], rows_v, sem).wait()
                    pltpu.sync_copy(rows_v, acc.at[sct_v], add=True)
                    return carry2

                lax.fori_loop(0, nb, do_batch, 0)
                pltpu.sync_copy(acc.at[pl.ds(base, CH)],
                                out.at[pl.ds(c * CH, CH)])
            return carry

        lax.fori_loop(0, rounds, do_round, 0)

    return seg


def _mm_relu(xx, w):
    BM = 1280
    G = xx.shape[0] // BM

    def body(x_ref, w_ref, o_ref):
        o_ref[...] = jnp.maximum(
            jnp.dot(x_ref[...], w_ref[...],
                    preferred_element_type=jnp.float32), 0.0)

    return pl.pallas_call(
        body,
        grid=(G,),
        in_specs=[pl.BlockSpec((BM, D), lambda i: (i, 0)),
                  pl.BlockSpec((D, D), lambda i: (0, 0))],
        out_specs=pl.BlockSpec((BM, D), lambda i: (i, 0)),
        out_shape=jax.ShapeDtypeStruct(xx.shape, jnp.float32),
    )(xx, w)


def _node_mm(agg2, h, em, wr2, ws, wm):
    BM = 1000
    G = N // BM

    def body(a_ref, h_ref, e_ref, wr_ref, ws_ref, wm_ref,
             o_ref, st_ref, acc_ref):
        i = pl.program_id(0)
        o = (jnp.dot(a_ref[...], wr_ref[...],
                     preferred_element_type=jnp.float32)
             + jnp.dot(h_ref[...], ws_ref[...],
                       preferred_element_type=jnp.float32)
             + jnp.dot(e_ref[...], wm_ref[...],
                       preferred_element_type=jnp.float32))
        o_ref[...] = o

        @pl.when(i == 0)
        def _():
            acc_ref[...] = jnp.zeros_like(acc_ref)

        acc_ref[0:1, :] = acc_ref[0:1, :] + jnp.sum(o, axis=0, keepdims=True)
        acc_ref[1:2, :] = acc_ref[1:2, :] + jnp.sum(o * o, axis=0,
                                                    keepdims=True)

        @pl.when(i == G - 1)
        def _():
            st_ref[...] = acc_ref[...]

    return pl.pallas_call(
        body,
        grid=(G,),
        in_specs=[pl.BlockSpec((BM, R * D), lambda i: (i, 0)),
                  pl.BlockSpec((BM, D), lambda i: (i, 0)),
                  pl.BlockSpec((BM, D), lambda i: (i, 0)),
                  pl.BlockSpec((R * D, D), lambda i: (0, 0)),
                  pl.BlockSpec((D, D), lambda i: (0, 0)),
                  pl.BlockSpec((D, D), lambda i: (0, 0))],
        out_specs=[pl.BlockSpec((BM, D), lambda i: (i, 0)),
                   pl.BlockSpec((8, D), lambda i: (0, 0))],
        out_shape=[jax.ShapeDtypeStruct((N, D), jnp.float32),
                   jax.ShapeDtypeStruct((8, D), jnp.float32)],
        scratch_shapes=[pltpu.VMEM((8, D), jnp.float32)],
    )(agg2, h, em, wr2, ws, wm)


def _bn_relu_skip(pre, stats, h, g2d, b2d):
    BM = 1000
    G = N // BM

    def body(p_ref, st_ref, h_ref, g_ref, b_ref, o_ref):
        mean = st_ref[0:1, :] * (1.0 / N)
        var = st_ref[1:2, :] * (1.0 / N) - mean * mean
        inv = lax.rsqrt(var + 1e-5)
        o = (p_ref[...] - mean) * inv * g_ref[...] + b_ref[...]
        o_ref[...] = jnp.maximum(o, 0.0) + h_ref[...]

    return pl.pallas_call(
        body,
        grid=(G,),
        in_specs=[pl.BlockSpec((BM, D), lambda i: (i, 0)),
                  pl.BlockSpec((8, D), lambda i: (0, 0)),
                  pl.BlockSpec((BM, D), lambda i: (i, 0)),
                  pl.BlockSpec((1, D), lambda i: (0, 0)),
                  pl.BlockSpec((1, D), lambda i: (0, 0))],
        out_specs=pl.BlockSpec((BM, D), lambda i: (i, 0)),
        out_shape=jax.ShapeDtypeStruct((N, D), jnp.float32),
    )(pre, stats, h, g2d, b2d)


def _prep(s, g, CH, C):
    loc = (s - (s // CH) * CH).astype(jnp.int32)
    bounds = jnp.searchsorted(
        s, jnp.arange(C + 1, dtype=jnp.int32) * CH).astype(jnp.int32)
    C1P = ((C + 1 + 16 + 7) // 8) * 8
    bounds = jnp.pad(bounds, (0, C1P - (C + 1)), constant_values=E)
    g = jnp.pad(g.astype(jnp.int32), (0, K))
    loc = jnp.pad(loc, (0, K))
    return g, loc, bounds


def kernel(x, edge_feat, Wr, Wself, Wm, We, gamma, beta,
           edge_index, edge_type, line_edge_index):
    src, dst = edge_index[0], edge_index[1]
    lsrc, ldst = line_edge_index[0], line_edge_index[1]
    rel = dst * R + edge_type

    p1 = jnp.argsort(ldst).astype(jnp.int32)
    g1, loc1, b1 = _prep(ldst[p1], lsrc[p1], 256, 1250)
    p2 = jnp.argsort(rel).astype(jnp.int32)
    s2 = rel[p2]
    g2, loc2, b2 = _prep(s2, src[p2], 112, 625)
    g3, loc3, b3 = _prep(s2 // R, p2, 80, 125)

    seg_line = _make_sc_segsum(256, 1250)
    seg_node = _make_sc_segsum(112, 625)
    seg_em = _make_sc_segsum(80, 125)
    z1 = jnp.zeros((256 + 8, D), jnp.float32)
    z2 = jnp.zeros((112 + 8, D), jnp.float32)
    z3 = jnp.zeros((80 + 8, D), jnp.float32)

    wr2 = Wr.reshape(L, R * D, D)
    h = x
    m = edge_feat
    for l in range(L):
        aggm = seg_line(m, g1, loc1, b1, z1)
        m = _mm_relu(aggm, We[l])
        agg = seg_node(h, g2, loc2, b2, z2)
        em = seg_em(m, g3, loc3, b3, z3)
        pre, stats = _node_mm(agg.reshape(N, R * D), h, em,
                              wr2[l], Wself[l], Wm[l])
        h = _bn_relu_skip(pre, stats, h, gamma[l].reshape(1, D),
                          beta[l].reshape(1, D))
    return h
